# baseline (speedup 1.0000x reference)
import functools

import jax
import jax.numpy as jnp
from jax import lax
from jax.experimental import pallas as pl
from jax.experimental.pallas import tpu as pltpu
from jax.experimental.pallas import tpu_sc as plsc

BATCH = 4096
SEQ = 200
DIM = 64
B = BATCH * SEQ
NC, NS = 2, 16
NW = NC * NS
BPW = B // NW
CHUNK = 320
NCHUNK = BPW // CHUNK
NBUF = 4
assert NCHUNK % NBUF == 0 and BPW % CHUNK == 0

_mesh = plsc.VectorSubcoreMesh(core_axis_name="c", subcore_axis_name="s")


@functools.partial(
    pl.kernel,
    out_type=jax.ShapeDtypeStruct((B, DIM), jnp.float32),
    mesh=_mesh,
    scratch_types=[
        pltpu.VMEM((BPW,), jnp.int32),
        pltpu.VMEM((NBUF, CHUNK, DIM), jnp.float32),
        [pltpu.SemaphoreType.DMA] * NBUF,
        [pltpu.SemaphoreType.DMA] * NBUF,
    ],
    compiler_params=pltpu.CompilerParams(use_tc_tiling_on_sc=False),
)
def _embed(idx_hbm, table_hbm, out_hbm, idx_v, rows_v, gsems, osems):
    wid = lax.axis_index("s") * NC + lax.axis_index("c")
    base = wid * BPW

    pltpu.sync_copy(idx_hbm.at[pl.ds(base, BPW)], idx_v)

    def gather(g, b):
        return pltpu.make_async_copy(
            table_hbm.at[idx_v.at[pl.ds(g * CHUNK, CHUNK)]],
            rows_v.at[b],
            gsems[b],
        )

    def writeback(g, b):
        return pltpu.make_async_copy(
            rows_v.at[b],
            out_hbm.at[pl.ds(base + g * CHUNK, CHUNK)],
            osems[b],
        )

    for b in range(NBUF):
        gather(b, b).start()

    @pl.loop(0, NCHUNK, step=NBUF)
    def _pipeline(g0):
        for b in range(NBUF):
            g = g0 + b
            gather(g, b).wait()
            writeback(g, b).start()
            nxt = g + NBUF

            @pl.when(nxt < NCHUNK)
            def _():
                writeback(g, b).wait()
                gather(nxt, b).start()

    for b in range(NBUF):
        writeback(NCHUNK - NBUF + b, b).wait()


def kernel(idxes, table):
    out = _embed(idxes.reshape(B), table)
    return out.reshape(BATCH, SEQ, DIM)


# speedup vs baseline: 1.0040x; 1.0040x over previous
import functools

import jax
import jax.numpy as jnp
from jax import lax
from jax.experimental import pallas as pl
from jax.experimental.pallas import tpu as pltpu
from jax.experimental.pallas import tpu_sc as plsc

BATCH = 4096
SEQ = 200
DIM = 64
B = BATCH * SEQ
NC, NS = 2, 16
NW = NC * NS
ROWS_PW = BATCH // NW
BPW = B // NW
CHUNKB = 2
CHUNK = CHUNKB * SEQ
NCHUNK = ROWS_PW // CHUNKB
NBUF = 4
assert NCHUNK % NBUF == 0 and ROWS_PW % CHUNKB == 0

_mesh = plsc.VectorSubcoreMesh(core_axis_name="c", subcore_axis_name="s")


@functools.partial(
    pl.kernel,
    out_type=jax.ShapeDtypeStruct((BATCH, SEQ, DIM), jnp.float32),
    mesh=_mesh,
    scratch_types=[
        pltpu.VMEM((BPW,), jnp.int32),
        pltpu.VMEM((NBUF, CHUNK, DIM), jnp.float32),
        [pltpu.SemaphoreType.DMA] * NBUF,
        [pltpu.SemaphoreType.DMA] * NBUF,
    ],
    compiler_params=pltpu.CompilerParams(use_tc_tiling_on_sc=False),
)
def _embed(idx_hbm, table_hbm, out_hbm, idx_v, rows_v, gsems, osems):
    wid = lax.axis_index("s") * NC + lax.axis_index("c")
    base = wid * BPW
    row0 = wid * ROWS_PW

    pltpu.sync_copy(idx_hbm.at[pl.ds(base, BPW)], idx_v)

    def gather(g, b):
        return pltpu.make_async_copy(
            table_hbm.at[idx_v.at[pl.ds(g * CHUNK, CHUNK)]],
            rows_v.at[b],
            gsems[b],
        )

    def writebacks(g, b):
        return [
            pltpu.make_async_copy(
                rows_v.at[b, pl.ds(j * SEQ, SEQ)],
                out_hbm.at[row0 + g * CHUNKB + j],
                osems[b],
            )
            for j in range(CHUNKB)
        ]

    for b in range(NBUF):
        gather(b, b).start()

    @pl.loop(0, NCHUNK, step=NBUF)
    def _pipeline(g0):
        for b in range(NBUF):
            g = g0 + b
            gather(g, b).wait()
            for wb in writebacks(g, b):
                wb.start()
            nxt = g + NBUF

            @pl.when(nxt < NCHUNK)
            def _():
                for wb in writebacks(g, b):
                    wb.wait()
                gather(nxt, b).start()

    for b in range(NBUF):
        for wb in writebacks(NCHUNK - NBUF + b, b):
            wb.wait()


def kernel(idxes, table):
    return _embed(idxes.reshape(B), table)
